# Initial kernel scaffold; baseline (speedup 1.0000x reference)
import functools

import jax
import jax.numpy as jnp
from jax import lax
from jax.experimental import pallas as pl
from jax.experimental.pallas import tpu as pltpu
from jax.experimental.pallas import tpu_sc as plsc

V = 10000
F = 128
NE = 320000
EPS = 1e-08

NC = 2
NS = 16
NW = NC * NS
VPAD = 10240
ROWS_PER_TILE = VPAD // NS
E_PER_W = NE // NW
CHUNK = 128
NFULL = E_PER_W // CHUNK
REM = E_PER_W - NFULL * CHUNK
DEGW = 8

_mesh = plsc.VectorSubcoreMesh(
    core_axis_name="c", subcore_axis_name="s", num_cores=NC, num_subcores=NS
)


@functools.partial(
    pl.kernel,
    out_type=jax.ShapeDtypeStruct((NC * VPAD, DEGW), jnp.float32),
    mesh=_mesh,
    scratch_types=[
        pltpu.VMEM((CHUNK,), jnp.int32),
        pltpu.VMEM((REM,), jnp.int32),
        pltpu.VMEM((CHUNK, DEGW), jnp.float32),
        pltpu.VMEM_SHARED((VPAD, DEGW), jnp.float32),
    ],
)
def _deg_kernel(dst_hbm, ones_hbm, zeros_hbm, out_hbm, idx_v, idx_r, ones_v,
                deg_sh):
    c = lax.axis_index("c")
    s = lax.axis_index("s")
    wid = c * NS + s
    ebase = wid * E_PER_W

    pltpu.sync_copy(ones_hbm, ones_v)
    pltpu.sync_copy(zeros_hbm, deg_sh.at[pl.ds(s * ROWS_PER_TILE,
                                               ROWS_PER_TILE)])
    plsc.subcore_barrier()

    def body(j, carry):
        base = pl.multiple_of(ebase + j * CHUNK, 8)
        pltpu.sync_copy(dst_hbm.at[pl.ds(base, CHUNK)], idx_v)
        pltpu.sync_copy(ones_v, deg_sh.at[idx_v], add=True)
        return carry

    lax.fori_loop(0, NFULL, body, 0)

    base = pl.multiple_of(ebase + NFULL * CHUNK, 8)
    pltpu.sync_copy(dst_hbm.at[pl.ds(base, REM)], idx_r)
    pltpu.sync_copy(ones_v.at[pl.ds(0, REM)], deg_sh.at[idx_r], add=True)

    plsc.subcore_barrier()
    pltpu.sync_copy(
        deg_sh.at[pl.ds(s * ROWS_PER_TILE, ROWS_PER_TILE)],
        out_hbm.at[pl.ds(c * VPAD + s * ROWS_PER_TILE, ROWS_PER_TILE)],
    )


@functools.partial(
    pl.kernel,
    out_type=jax.ShapeDtypeStruct((NC * VPAD, F), jnp.float32),
    mesh=_mesh,
    scratch_types=[
        pltpu.VMEM((CHUNK,), jnp.int32),
        pltpu.VMEM((CHUNK,), jnp.int32),
        pltpu.VMEM((REM,), jnp.int32),
        pltpu.VMEM((REM,), jnp.int32),
        pltpu.VMEM((CHUNK, F), jnp.float32),
        pltpu.VMEM((REM, F), jnp.float32),
        pltpu.VMEM_SHARED((VPAD, F), jnp.float32),
        pltpu.SemaphoreType.DMA,
    ],
)
def _segsum_kernel(xp_hbm, src_hbm, dst_hbm, zeros_hbm, out_hbm,
                   sidx, didx, sidx_r, didx_r, rows, rows_r, acc_sh, sem):
    c = lax.axis_index("c")
    s = lax.axis_index("s")
    wid = c * NS + s
    ebase = wid * E_PER_W

    for k in range(ROWS_PER_TILE // CHUNK):
        pltpu.sync_copy(
            zeros_hbm,
            acc_sh.at[pl.ds(s * ROWS_PER_TILE + k * CHUNK, CHUNK)],
        )
    plsc.subcore_barrier()

    def body(j, carry):
        base = pl.multiple_of(ebase + j * CHUNK, 8)
        pltpu.sync_copy(src_hbm.at[pl.ds(base, CHUNK)], sidx)
        pltpu.sync_copy(dst_hbm.at[pl.ds(base, CHUNK)], didx)
        pltpu.async_copy(xp_hbm.at[sidx], rows, sem).wait()
        pltpu.sync_copy(rows, acc_sh.at[didx], add=True)
        return carry

    lax.fori_loop(0, NFULL, body, 0)

    base = pl.multiple_of(ebase + NFULL * CHUNK, 8)
    pltpu.sync_copy(src_hbm.at[pl.ds(base, REM)], sidx_r)
    pltpu.sync_copy(dst_hbm.at[pl.ds(base, REM)], didx_r)
    pltpu.async_copy(xp_hbm.at[sidx_r], rows_r, sem).wait()
    pltpu.sync_copy(rows_r, acc_sh.at[didx_r], add=True)

    plsc.subcore_barrier()
    pltpu.sync_copy(
        acc_sh.at[pl.ds(s * ROWS_PER_TILE, ROWS_PER_TILE)],
        out_hbm.at[pl.ds(c * VPAD + s * ROWS_PER_TILE, ROWS_PER_TILE)],
    )


def _scale_body(dg0_ref, dg1_ref, x_ref, xp_ref):
    deg = dg0_ref[:, :1] + dg1_ref[:, :1] + 1.0
    dinv = lax.rsqrt(deg)
    xp_ref[...] = x_ref[...] * dinv


def _scale(dg0, dg1, x):
    blk = 1000
    grid = (V // blk,)
    return pl.pallas_call(
        _scale_body,
        grid=grid,
        in_specs=[
            pl.BlockSpec((blk, DEGW), lambda i: (i, 0)),
            pl.BlockSpec((blk, DEGW), lambda i: (i, 0)),
            pl.BlockSpec((blk, F), lambda i: (i, 0)),
        ],
        out_specs=pl.BlockSpec((blk, F), lambda i: (i, 0)),
        out_shape=jax.ShapeDtypeStruct((V, F), jnp.float32),
    )(dg0, dg1, x)


def _heads_body(a0_ref, a1_ref, xp_ref, dg0_ref, dg1_ref, wg_ref, bg_ref,
                wl_ref, bl_ref, ws_ref, bs_ref, eps_ref,
                z_ref, mu_ref, lv_ref):
    deg = dg0_ref[:, :1] + dg1_ref[:, :1] + 1.0
    dinv = lax.rsqrt(deg)
    y = dinv * (a0_ref[...] + a1_ref[...] + xp_ref[...])
    g = (
        lax.dot_general(y, wg_ref[...], (((1,), (0,)), ((), ())),
                        precision=lax.Precision.HIGHEST,
                        preferred_element_type=jnp.float32)
        + bg_ref[...]
    )
    mu = (
        lax.dot_general(g, wl_ref[...], (((1,), (1,)), ((), ())),
                        precision=lax.Precision.HIGHEST,
                        preferred_element_type=jnp.float32)
        + bl_ref[...]
    )
    t = (
        lax.dot_general(g, ws_ref[...], (((1,), (1,)), ((), ())),
                        precision=lax.Precision.HIGHEST,
                        preferred_element_type=jnp.float32)
        + bs_ref[...]
    )
    sp = jnp.maximum(t, 0.0) + jnp.log1p(jnp.exp(-jnp.abs(t))) + EPS
    mu_ref[...] = mu
    z_ref[...] = mu + sp * eps_ref[...]
    lv_ref[...] = 2.0 * jnp.log(sp)


def _heads(a0, a1, xp, dg0, dg1, W_gcn, b_gcn, W_loc, b_loc, W_std, b_std,
           eps):
    blk = 1000
    grid = (V // blk,)
    row = lambda i: (i, 0)
    full = lambda i: (0, 0)
    out_sds = jax.ShapeDtypeStruct((V, F), jnp.float32)
    return pl.pallas_call(
        _heads_body,
        grid=grid,
        in_specs=[
            pl.BlockSpec((blk, F), row),
            pl.BlockSpec((blk, F), row),
            pl.BlockSpec((blk, F), row),
            pl.BlockSpec((blk, DEGW), row),
            pl.BlockSpec((blk, DEGW), row),
            pl.BlockSpec((F, F), full),
            pl.BlockSpec((1, F), full),
            pl.BlockSpec((F, F), full),
            pl.BlockSpec((1, F), full),
            pl.BlockSpec((F, F), full),
            pl.BlockSpec((1, F), full),
            pl.BlockSpec((blk, F), row),
        ],
        out_specs=[
            pl.BlockSpec((blk, F), row),
            pl.BlockSpec((blk, F), row),
            pl.BlockSpec((blk, F), row),
        ],
        out_shape=[out_sds, out_sds, out_sds],
    )(a0, a1, xp, dg0, dg1, W_gcn, b_gcn, W_loc, b_loc, W_std, b_std, eps)


def kernel(edge_index, vrepr, W_gcn, b_gcn, W_loc, b_loc, W_std, b_std, eps):
    src = edge_index[0]
    dst = edge_index[1]

    ones8 = jnp.ones((CHUNK, DEGW), jnp.float32)
    zeros8 = jnp.zeros((ROWS_PER_TILE, DEGW), jnp.float32)
    zerosF = jnp.zeros((CHUNK, F), jnp.float32)

    degp = _deg_kernel(dst, ones8, zeros8)
    dg0 = degp[:V]
    dg1 = degp[VPAD:VPAD + V]

    xp = _scale(dg0, dg1, vrepr)

    accp = _segsum_kernel(xp, src, dst, zerosF)
    a0 = accp[:V]
    a1 = accp[VPAD:VPAD + V]

    z, mu, lv = _heads(a0, a1, xp, dg0, dg1, W_gcn,
                       b_gcn.reshape(1, F), W_loc, b_loc.reshape(1, F),
                       W_std, b_std.reshape(1, F), eps)
    return (z, mu, lv)


# speedup vs baseline: 21.8553x; 21.8553x over previous
import functools

import jax
import jax.numpy as jnp
from jax import lax
from jax.experimental import pallas as pl
from jax.experimental.pallas import tpu as pltpu
from jax.experimental.pallas import tpu_sc as plsc

V = 10000
F = 128
NE = 320000
EPS = 1e-08

NC = 2
NS = 16
NW = NC * NS
VPAD = 10240
RPT = VPAD // NS
EPW = NE // NW
CH = 80
NCH = EPW // CH
NGRP = (NCH - 1) // 4
CHS = 128
NFULL = EPW // CHS
REM = EPW - NFULL * CHS
DEGW = 128


@functools.cache
def _deg_kernel():
    mesh = plsc.VectorSubcoreMesh(core_axis_name="c", subcore_axis_name="s",
                                  num_cores=NC, num_subcores=NS)

    @functools.partial(
        pl.kernel,
        out_type=jax.ShapeDtypeStruct((NC * VPAD, DEGW), jnp.float32),
        mesh=mesh,
        scratch_types=[
            pltpu.VMEM((CH,), jnp.int32),
            pltpu.VMEM((CH,), jnp.int32),
            pltpu.VMEM((CH,), jnp.int32),
            pltpu.VMEM((CH,), jnp.int32),
            pltpu.VMEM((CH, DEGW), jnp.float32),
            pltpu.VMEM_SHARED((VPAD, DEGW), jnp.float32),
            pltpu.SemaphoreType.DMA,
        ],
    )
    def deg_k(dst_hbm, ones_hbm, zeros_hbm, out_hbm, i0, i1, i2, i3,
              ones_v, deg_sh, sem):
        c = lax.axis_index("c")
        s = lax.axis_index("s")
        ebase = (c * NS + s) * EPW

        pltpu.sync_copy(ones_hbm, ones_v)
        pltpu.sync_copy(zeros_hbm, deg_sh.at[pl.ds(s * RPT, RPT)])
        plsc.subcore_barrier()

        bufs = (i0, i1, i2, i3)

        def body(g, carry):
            for k in range(4):
                base = pl.multiple_of(ebase + (g * 4 + k) * CH, 8)
                pltpu.sync_copy(dst_hbm.at[pl.ds(base, CH)], bufs[k])
                pltpu.async_copy(ones_v, deg_sh.at[bufs[k]], sem, add=True)
            for k in range(4):
                pltpu.make_async_copy(ones_v, deg_sh.at[i0], sem).wait()
            return carry

        lax.fori_loop(0, NGRP, body, 0)
        base = pl.multiple_of(ebase + (NCH - 1) * CH, 8)
        pltpu.sync_copy(dst_hbm.at[pl.ds(base, CH)], i0)
        pltpu.sync_copy(ones_v, deg_sh.at[i0], add=True)

        plsc.subcore_barrier()
        pltpu.sync_copy(deg_sh.at[pl.ds(s * RPT, RPT)],
                        out_hbm.at[pl.ds(c * VPAD + s * RPT, RPT)])

    return deg_k


@functools.cache
def _segsum_kernel():
    mesh = plsc.VectorSubcoreMesh(core_axis_name="c", subcore_axis_name="s",
                                  num_cores=NC, num_subcores=NS)

    @functools.partial(
        pl.kernel,
        out_type=jax.ShapeDtypeStruct((NC * VPAD, F), jnp.float32),
        mesh=mesh,
        scratch_types=[
            pltpu.VMEM((CHS,), jnp.int32),
            pltpu.VMEM((CHS,), jnp.int32),
            pltpu.VMEM((CHS,), jnp.int32),
            pltpu.VMEM((CHS,), jnp.int32),
            pltpu.VMEM((REM,), jnp.int32),
            pltpu.VMEM((REM,), jnp.int32),
            pltpu.VMEM((CHS, F), jnp.float32),
            pltpu.VMEM((CHS, F), jnp.float32),
            pltpu.VMEM((REM, F), jnp.float32),
            pltpu.VMEM_SHARED((VPAD, F), jnp.float32),
            pltpu.SemaphoreType.DMA,
            pltpu.SemaphoreType.DMA,
        ],
    )
    def segsum_k(xp_hbm, src_hbm, dst_hbm, zeros_hbm, out_hbm,
                 s0, s1, d0, d1, sr, dr, r0, r1, rr, acc_sh, m0, m1):
        c = lax.axis_index("c")
        s = lax.axis_index("s")
        ebase = (c * NS + s) * EPW

        for k in range(RPT // CHS):
            pltpu.sync_copy(zeros_hbm,
                            acc_sh.at[pl.ds(s * RPT + k * CHS, CHS)])
        plsc.subcore_barrier()

        sb = (s0, s1)
        db = (d0, d1)
        rb = (r0, r1)
        sems = (m0, m1)

        def body(g, carry):
            for k in range(2):
                base = pl.multiple_of(ebase + (g * 2 + k) * CHS, 8)
                pltpu.sync_copy(src_hbm.at[pl.ds(base, CHS)], sb[k])
                pltpu.async_copy(xp_hbm.at[sb[k]], rb[k], sems[k])
            for k in range(2):
                base = pl.multiple_of(ebase + (g * 2 + k) * CHS, 8)
                pltpu.sync_copy(dst_hbm.at[pl.ds(base, CHS)], db[k])
                pltpu.make_async_copy(xp_hbm.at[sb[k]], rb[k],
                                      sems[k]).wait()
                pltpu.sync_copy(rb[k], acc_sh.at[db[k]], add=True)
            return carry

        lax.fori_loop(0, NFULL // 2, body, 0)

        base = pl.multiple_of(ebase + NFULL * CHS, 8)
        pltpu.sync_copy(src_hbm.at[pl.ds(base, REM)], sr)
        pltpu.sync_copy(dst_hbm.at[pl.ds(base, REM)], dr)
        pltpu.async_copy(xp_hbm.at[sr], rr, m0).wait()
        pltpu.sync_copy(rr, acc_sh.at[dr], add=True)

        plsc.subcore_barrier()
        pltpu.sync_copy(acc_sh.at[pl.ds(s * RPT, RPT)],
                        out_hbm.at[pl.ds(c * VPAD + s * RPT, RPT)])

    return segsum_k


def _scale_body(dg0_ref, dg1_ref, x_ref, xp_ref):
    deg = dg0_ref[:, :1] + dg1_ref[:, :1] + 1.0
    dinv = lax.rsqrt(deg)
    xp_ref[...] = x_ref[...] * dinv


def _scale(dg0, dg1, x):
    blk = 1000
    grid = (V // blk,)
    return pl.pallas_call(
        _scale_body,
        grid=grid,
        in_specs=[
            pl.BlockSpec((blk, DEGW), lambda i: (i, 0)),
            pl.BlockSpec((blk, DEGW), lambda i: (i, 0)),
            pl.BlockSpec((blk, F), lambda i: (i, 0)),
        ],
        out_specs=pl.BlockSpec((blk, F), lambda i: (i, 0)),
        out_shape=jax.ShapeDtypeStruct((V, F), jnp.float32),
    )(dg0, dg1, x)


def _heads_body(a0_ref, a1_ref, xp_ref, dg0_ref, dg1_ref, wg_ref, bg_ref,
                wl_ref, bl_ref, ws_ref, bs_ref, eps_ref,
                z_ref, mu_ref, lv_ref):
    deg = dg0_ref[:, :1] + dg1_ref[:, :1] + 1.0
    dinv = lax.rsqrt(deg)
    y = dinv * (a0_ref[...] + a1_ref[...] + xp_ref[...])
    g = (
        lax.dot_general(y, wg_ref[...], (((1,), (0,)), ((), ())),
                        precision=lax.Precision.HIGHEST,
                        preferred_element_type=jnp.float32)
        + bg_ref[...]
    )
    mu = (
        lax.dot_general(g, wl_ref[...], (((1,), (1,)), ((), ())),
                        precision=lax.Precision.HIGHEST,
                        preferred_element_type=jnp.float32)
        + bl_ref[...]
    )
    t = (
        lax.dot_general(g, ws_ref[...], (((1,), (1,)), ((), ())),
                        precision=lax.Precision.HIGHEST,
                        preferred_element_type=jnp.float32)
        + bs_ref[...]
    )
    sp = jnp.maximum(t, 0.0) + jnp.log1p(jnp.exp(-jnp.abs(t))) + EPS
    mu_ref[...] = mu
    z_ref[...] = mu + sp * eps_ref[...]
    lv_ref[...] = 2.0 * jnp.log(sp)


def _heads(a0, a1, xp, dg0, dg1, W_gcn, b_gcn, W_loc, b_loc, W_std, b_std,
           eps):
    blk = 1000
    grid = (V // blk,)
    row = lambda i: (i, 0)
    full = lambda i: (0, 0)
    out_sds = jax.ShapeDtypeStruct((V, F), jnp.float32)
    return pl.pallas_call(
        _heads_body,
        grid=grid,
        in_specs=[
            pl.BlockSpec((blk, F), row),
            pl.BlockSpec((blk, F), row),
            pl.BlockSpec((blk, F), row),
            pl.BlockSpec((blk, DEGW), row),
            pl.BlockSpec((blk, DEGW), row),
            pl.BlockSpec((F, F), full),
            pl.BlockSpec((1, F), full),
            pl.BlockSpec((F, F), full),
            pl.BlockSpec((1, F), full),
            pl.BlockSpec((F, F), full),
            pl.BlockSpec((1, F), full),
            pl.BlockSpec((blk, F), row),
        ],
        out_specs=[
            pl.BlockSpec((blk, F), row),
            pl.BlockSpec((blk, F), row),
            pl.BlockSpec((blk, F), row),
        ],
        out_shape=[out_sds, out_sds, out_sds],
    )(a0, a1, xp, dg0, dg1, W_gcn, b_gcn, W_loc, b_loc, W_std, b_std, eps)


def kernel(edge_index, vrepr, W_gcn, b_gcn, W_loc, b_loc, W_std, b_std, eps):
    src = edge_index[0]
    dst = edge_index[1]

    ones16 = jnp.ones((CH, DEGW), jnp.float32)
    zeros16 = jnp.zeros((RPT, DEGW), jnp.float32)
    zerosF = jnp.zeros((CHS, F), jnp.float32)

    degp = _deg_kernel()(dst, ones16, zeros16)
    dg0 = degp[:V]
    dg1 = degp[VPAD:VPAD + V]

    xp = _scale(dg0, dg1, vrepr)

    accp = _segsum_kernel()(xp, src, dst, zerosF)
    a0 = accp[:V]
    a1 = accp[VPAD:VPAD + V]

    z, mu, lv = _heads(a0, a1, xp, dg0, dg1, W_gcn,
                       b_gcn.reshape(1, F), W_loc, b_loc.reshape(1, F),
                       W_std, b_std.reshape(1, F), eps)
    return (z, mu, lv)


# speedup vs baseline: 26.0927x; 1.1939x over previous
import functools

import jax
import jax.numpy as jnp
from jax import lax
from jax.experimental import pallas as pl
from jax.experimental.pallas import tpu as pltpu
from jax.experimental.pallas import tpu_sc as plsc

V = 10000
F = 128
NE = 320000
EPS = 1e-08

NC = 2
NS = 16
NW = NC * NS
VPAD = 10240
RPT = VPAD // NS
EPW = NE // NW
CHS = 128
NFULL = EPW // CHS
REM = EPW - NFULL * CHS
DEGW = 128


@functools.cache
def _deg_kernel():
    mesh = plsc.VectorSubcoreMesh(core_axis_name="c", subcore_axis_name="s",
                                  num_cores=NC, num_subcores=NS)

    @functools.partial(
        pl.kernel,
        out_type=jax.ShapeDtypeStruct((NC * VPAD, DEGW), jnp.float32),
        mesh=mesh,
        scratch_types=[
            pltpu.VMEM((CHS,), jnp.int32),
            pltpu.VMEM((CHS,), jnp.int32),
            pltpu.VMEM((REM,), jnp.int32),
            pltpu.VMEM((CHS, DEGW), jnp.float32),
            pltpu.VMEM_SHARED((VPAD, DEGW), jnp.float32),
            pltpu.SemaphoreType.DMA,
        ],
    )
    def deg_k(dst_hbm, ones_hbm, zeros_hbm, out_hbm, i0, i1, ir,
              ones_v, deg_sh, sem):
        c = lax.axis_index("c")
        s = lax.axis_index("s")
        ebase = (c * NS + s) * EPW

        pltpu.sync_copy(ones_hbm, ones_v)
        for k in range(RPT // CHS):
            pltpu.sync_copy(zeros_hbm,
                            deg_sh.at[pl.ds(s * RPT + k * CHS, CHS)])
        plsc.subcore_barrier()

        bufs = (i0, i1)

        def body(g, carry):
            for k in range(2):
                base = pl.multiple_of(ebase + (g * 2 + k) * CHS, 8)
                pltpu.sync_copy(dst_hbm.at[pl.ds(base, CHS)], bufs[k])
                pltpu.async_copy(ones_v, deg_sh.at[bufs[k]], sem, add=True)
            for k in range(2):
                pltpu.make_async_copy(ones_v, deg_sh.at[i0], sem).wait()
            return carry

        lax.fori_loop(0, NFULL // 2, body, 0)
        base = pl.multiple_of(ebase + NFULL * CHS, 8)
        pltpu.sync_copy(dst_hbm.at[pl.ds(base, REM)], ir)
        pltpu.sync_copy(ones_v.at[pl.ds(0, REM)], deg_sh.at[ir], add=True)

        plsc.subcore_barrier()
        pltpu.sync_copy(deg_sh.at[pl.ds(s * RPT, RPT)],
                        out_hbm.at[pl.ds(c * VPAD + s * RPT, RPT)])

    return deg_k


@functools.cache
def _segsum_kernel():
    mesh = plsc.VectorSubcoreMesh(core_axis_name="c", subcore_axis_name="s",
                                  num_cores=NC, num_subcores=NS)

    @functools.partial(
        pl.kernel,
        out_type=jax.ShapeDtypeStruct((NC * VPAD, F), jnp.float32),
        mesh=mesh,
        scratch_types=[
            pltpu.VMEM((EPW,), jnp.int32),
            pltpu.VMEM((CHS,), jnp.int32),
            pltpu.VMEM((CHS,), jnp.int32),
            pltpu.VMEM((REM,), jnp.int32),
            pltpu.VMEM((CHS, F), jnp.float32),
            pltpu.VMEM((CHS, F), jnp.float32),
            pltpu.VMEM_SHARED((VPAD, F), jnp.float32),
            pltpu.SemaphoreType.DMA,
            pltpu.SemaphoreType.DMA,
        ],
    )
    def segsum_k(xp_hbm, src_hbm, dst_hbm, zeros_hbm, out_hbm,
                 sidx_all, d0, d1, dr, r0, r1, acc_sh, m0, m1):
        c = lax.axis_index("c")
        s = lax.axis_index("s")
        ebase = (c * NS + s) * EPW

        pltpu.sync_copy(src_hbm.at[pl.ds(ebase, EPW)], sidx_all)
        for k in range(RPT // CHS):
            pltpu.sync_copy(zeros_hbm,
                            acc_sh.at[pl.ds(s * RPT + k * CHS, CHS)])
        plsc.subcore_barrier()

        db = (d0, d1)
        rb = (r0, r1)
        sems = (m0, m1)

        def fire(j, k):
            off = pl.multiple_of(j * CHS, 8)
            pltpu.async_copy(xp_hbm.at[sidx_all.at[pl.ds(off, CHS)]],
                             rb[k], sems[k])

        def drain(j, k):
            base = pl.multiple_of(ebase + j * CHS, 8)
            pltpu.sync_copy(dst_hbm.at[pl.ds(base, CHS)], db[k])
            pltpu.make_async_copy(xp_hbm.at[sidx_all.at[pl.ds(0, CHS)]],
                                  rb[k], sems[k]).wait()
            pltpu.sync_copy(rb[k], acc_sh.at[db[k]], add=True)

        fire(0, 0)
        fire(1, 1)

        def body(g, carry):
            for k in range(2):
                drain(g * 2 + k, k)
                fire(g * 2 + k + 2, k)
            return carry

        lax.fori_loop(0, NFULL // 2 - 1, body, 0)
        drain(NFULL - 2, 0)
        drain(NFULL - 1, 1)

        base = pl.multiple_of(ebase + NFULL * CHS, 8)
        pltpu.sync_copy(dst_hbm.at[pl.ds(base, REM)], dr)
        off = pl.multiple_of(NFULL * CHS, 8)
        pltpu.async_copy(xp_hbm.at[sidx_all.at[pl.ds(off, REM)]],
                         r0.at[pl.ds(0, REM)], m0).wait()
        pltpu.sync_copy(r0.at[pl.ds(0, REM)], acc_sh.at[dr], add=True)

        plsc.subcore_barrier()
        pltpu.sync_copy(acc_sh.at[pl.ds(s * RPT, RPT)],
                        out_hbm.at[pl.ds(c * VPAD + s * RPT, RPT)])

    return segsum_k


def _scale_body(dg0_ref, dg1_ref, x_ref, xp_ref):
    deg = dg0_ref[:, :1] + dg1_ref[:, :1] + 1.0
    dinv = lax.rsqrt(deg)
    xp_ref[...] = x_ref[...] * dinv


def _scale(dg0, dg1, x):
    blk = 1000
    grid = (V // blk,)
    return pl.pallas_call(
        _scale_body,
        grid=grid,
        in_specs=[
            pl.BlockSpec((blk, DEGW), lambda i: (i, 0)),
            pl.BlockSpec((blk, DEGW), lambda i: (i, 0)),
            pl.BlockSpec((blk, F), lambda i: (i, 0)),
        ],
        out_specs=pl.BlockSpec((blk, F), lambda i: (i, 0)),
        out_shape=jax.ShapeDtypeStruct((V, F), jnp.float32),
    )(dg0, dg1, x)


def _heads_body(a0_ref, a1_ref, xp_ref, dg0_ref, dg1_ref, wg_ref, bg_ref,
                wl_ref, bl_ref, ws_ref, bs_ref, eps_ref,
                z_ref, mu_ref, lv_ref):
    deg = dg0_ref[:, :1] + dg1_ref[:, :1] + 1.0
    dinv = lax.rsqrt(deg)
    y = dinv * (a0_ref[...] + a1_ref[...] + xp_ref[...])
    g = (
        lax.dot_general(y, wg_ref[...], (((1,), (0,)), ((), ())),
                        precision=lax.Precision.HIGHEST,
                        preferred_element_type=jnp.float32)
        + bg_ref[...]
    )
    mu = (
        lax.dot_general(g, wl_ref[...], (((1,), (1,)), ((), ())),
                        precision=lax.Precision.HIGHEST,
                        preferred_element_type=jnp.float32)
        + bl_ref[...]
    )
    t = (
        lax.dot_general(g, ws_ref[...], (((1,), (1,)), ((), ())),
                        precision=lax.Precision.HIGHEST,
                        preferred_element_type=jnp.float32)
        + bs_ref[...]
    )
    sp = jnp.maximum(t, 0.0) + jnp.log1p(jnp.exp(-jnp.abs(t))) + EPS
    mu_ref[...] = mu
    z_ref[...] = mu + sp * eps_ref[...]
    lv_ref[...] = 2.0 * jnp.log(sp)


def _heads(a0, a1, xp, dg0, dg1, W_gcn, b_gcn, W_loc, b_loc, W_std, b_std,
           eps):
    blk = 1000
    grid = (V // blk,)
    row = lambda i: (i, 0)
    full = lambda i: (0, 0)
    out_sds = jax.ShapeDtypeStruct((V, F), jnp.float32)
    return pl.pallas_call(
        _heads_body,
        grid=grid,
        in_specs=[
            pl.BlockSpec((blk, F), row),
            pl.BlockSpec((blk, F), row),
            pl.BlockSpec((blk, F), row),
            pl.BlockSpec((blk, DEGW), row),
            pl.BlockSpec((blk, DEGW), row),
            pl.BlockSpec((F, F), full),
            pl.BlockSpec((1, F), full),
            pl.BlockSpec((F, F), full),
            pl.BlockSpec((1, F), full),
            pl.BlockSpec((F, F), full),
            pl.BlockSpec((1, F), full),
            pl.BlockSpec((blk, F), row),
        ],
        out_specs=[
            pl.BlockSpec((blk, F), row),
            pl.BlockSpec((blk, F), row),
            pl.BlockSpec((blk, F), row),
        ],
        out_shape=[out_sds, out_sds, out_sds],
    )(a0, a1, xp, dg0, dg1, W_gcn, b_gcn, W_loc, b_loc, W_std, b_std, eps)


def kernel(edge_index, vrepr, W_gcn, b_gcn, W_loc, b_loc, W_std, b_std, eps):
    src = edge_index[0]
    dst = edge_index[1]

    zerosF = jnp.zeros((CHS, F), jnp.float32)
    onesD = jnp.ones((CHS, DEGW), jnp.float32)

    degp = _deg_kernel()(dst, onesD, zerosF)
    dg0 = degp[:V]
    dg1 = degp[VPAD:VPAD + V]

    xp = _scale(dg0, dg1, vrepr)

    accp = _segsum_kernel()(xp, src, dst, zerosF)
    a0 = accp[:V]
    a1 = accp[VPAD:VPAD + V]

    z, mu, lv = _heads(a0, a1, xp, dg0, dg1, W_gcn,
                       b_gcn.reshape(1, F), W_loc, b_loc.reshape(1, F),
                       W_std, b_std.reshape(1, F), eps)
    return (z, mu, lv)
